# baseline (speedup 1.0000x reference)
import functools

import jax
import jax.numpy as jnp
from jax import lax
from jax.experimental import pallas as pl
from jax.experimental.pallas import tpu as pltpu
from jax.experimental.pallas import tpu_sc as plsc

VOCAB = 100000
EMBED = 768
SEQ = 2048
BATCH = 4

NUM_CORES = 2
NUM_SUBCORES = 16
NW = NUM_CORES * NUM_SUBCORES
ROWS = BATCH * SEQ
R_PER_W = ROWS // NW
C = 32
NCHUNK = R_PER_W // C
NSLOT = 2
W_PER_B = SEQ // R_PER_W
LANES = 16
NVEC = EMBED // LANES

_mesh = plsc.VectorSubcoreMesh(core_axis_name="c", subcore_axis_name="s")


@functools.partial(
    pl.kernel,
    mesh=_mesh,
    out_type=jax.ShapeDtypeStruct((BATCH, SEQ, EMBED), jnp.float32),
    scratch_types=(
        [pltpu.VMEM((R_PER_W,), jnp.int32)] * 2
        + [pltpu.VMEM((C, EMBED), jnp.float32)] * (2 * NSLOT)
        + [pltpu.SemaphoreType.DMA] * (2 * NSLOT + 1)
    ),
)
def _embed_sc(tok_hbm, pos_hbm, tok_table, pos_table, out_hbm, *scratch):
    tok_idx, pos_idx = scratch[0], scratch[1]
    bufs_a = list(scratch[2:2 + NSLOT])
    bufs_b = list(scratch[2 + NSLOT:2 + 2 * NSLOT])
    sems_g = list(scratch[2 + 2 * NSLOT:2 + 3 * NSLOT])
    sems_o = list(scratch[2 + 3 * NSLOT:2 + 4 * NSLOT])
    sem_i = scratch[2 + 4 * NSLOT]

    wid = lax.axis_index("s") * NUM_CORES + lax.axis_index("c")
    b = wid // W_PER_B
    s0 = (wid % W_PER_B) * R_PER_W
    icp_t = pltpu.async_copy(tok_hbm.at[b, pl.ds(s0, R_PER_W)], tok_idx, sem_i)
    icp_p = pltpu.async_copy(pos_hbm.at[b, pl.ds(s0, R_PER_W)], pos_idx, sem_i)
    icp_t.wait()
    icp_p.wait()


    def fire(c):
        s = c % NSLOT
        ga = pltpu.async_copy(
            tok_table.at[tok_idx.at[pl.ds(c * C, C)]], bufs_a[s], sems_g[s])
        gb = pltpu.async_copy(
            pos_table.at[pos_idx.at[pl.ds(c * C, C)]], bufs_b[s], sems_g[s])
        return ga, gb

    gcp = [None] * NSLOT
    ocp = [None] * NSLOT
    for c in range(NSLOT - 1):
        gcp[c % NSLOT] = fire(c)
    for c in range(NCHUNK):
        s = c % NSLOT
        nxt = c + NSLOT - 1
        if nxt < NCHUNK:
            sn = nxt % NSLOT
            if ocp[sn] is not None:
                ocp[sn].wait()
            gcp[sn] = fire(nxt)
        ga, gb = gcp[s]
        ga.wait()
        gb.wait()

        def row(i, carry, s=s):
            for r in range(4):
                for j in range(NVEC):
                    x = bufs_b[s][4 * i + r, pl.ds(j * LANES, LANES)]
                    plsc.addupdate(
                        bufs_a[s].at[4 * i + r, pl.ds(j * LANES, LANES)], x)
            return carry

        lax.fori_loop(0, C // 4, row, 0)
        ocp[s] = pltpu.async_copy(
            bufs_a[s], out_hbm.at[b, pl.ds(s0 + c * C, C)], sems_o[s])
    for s in range(NSLOT):
        if ocp[s] is not None:
            ocp[s].wait()


def kernel(tokens, positions, tok_table, pos_table):
    return _embed_sc(tokens, positions, tok_table, pos_table)


# speedup vs baseline: 1.0638x; 1.0638x over previous
import functools

import jax
import jax.numpy as jnp
from jax import lax
from jax.experimental import pallas as pl
from jax.experimental.pallas import tpu as pltpu
from jax.experimental.pallas import tpu_sc as plsc

VOCAB = 100000
EMBED = 768
SEQ = 2048
BATCH = 4

NUM_CORES = 2
NUM_SUBCORES = 16
NW = NUM_CORES * NUM_SUBCORES
ROWS = BATCH * SEQ
R_PER_W = ROWS // NW
C = 32
NCHUNK = R_PER_W // C
NSLOT = 2
W_PER_B = SEQ // R_PER_W
LANES = 16
NVEC = EMBED // LANES

_mesh = plsc.VectorSubcoreMesh(core_axis_name="c", subcore_axis_name="s")


@functools.partial(
    pl.kernel,
    mesh=_mesh,
    out_type=jax.ShapeDtypeStruct((BATCH, SEQ, EMBED), jnp.float32),
    scratch_types=(
        [pltpu.VMEM((R_PER_W,), jnp.int32)] * 2
        + [pltpu.VMEM((C, EMBED), jnp.float32)] * (2 * NSLOT)
        + [pltpu.SemaphoreType.DMA] * (2 * NSLOT + 1)
    ),
)
def _embed_sc(tok_hbm, pos_hbm, tok_table, pos_table, out_hbm, *scratch):
    tok_idx, pos_idx = scratch[0], scratch[1]
    bufs_a = list(scratch[2:2 + NSLOT])
    bufs_b = list(scratch[2 + NSLOT:2 + 2 * NSLOT])
    sems_g = list(scratch[2 + 2 * NSLOT:2 + 3 * NSLOT])
    sems_o = list(scratch[2 + 3 * NSLOT:2 + 4 * NSLOT])
    sem_i = scratch[2 + 4 * NSLOT]

    wid = lax.axis_index("s") * NUM_CORES + lax.axis_index("c")
    b = wid // W_PER_B
    s0 = (wid % W_PER_B) * R_PER_W
    icp_t = pltpu.async_copy(tok_hbm.at[b, pl.ds(s0, R_PER_W)], tok_idx, sem_i)
    icp_p = pltpu.async_copy(pos_hbm.at[b, pl.ds(s0, R_PER_W)], pos_idx, sem_i)
    icp_t.wait()
    icp_p.wait()


    def fire(c):
        s = c % NSLOT
        ga = pltpu.async_copy(
            tok_table.at[tok_idx.at[pl.ds(c * C, C)]], bufs_a[s], sems_g[s])
        gb = pltpu.async_copy(
            pos_table.at[pos_idx.at[pl.ds(c * C, C)]], bufs_b[s], sems_g[s])
        return ga, gb

    gcp = [None] * NSLOT
    ocp = [None] * NSLOT
    for c in range(NSLOT - 1):
        gcp[c % NSLOT] = fire(c)
    for c in range(NCHUNK):
        s = c % NSLOT
        nxt = c + NSLOT - 1
        if nxt < NCHUNK:
            sn = nxt % NSLOT
            if ocp[sn] is not None:
                ocp[sn].wait()
            gcp[sn] = fire(nxt)
        ga, gb = gcp[s]
        ga.wait()
        gb.wait()

        def row(i, carry, s=s):
            for r in range(2):
                for j in range(NVEC):
                    x = bufs_b[s][2 * i + r, pl.ds(j * LANES, LANES)]
                    plsc.addupdate(
                        bufs_a[s].at[2 * i + r, pl.ds(j * LANES, LANES)], x)
            return carry

        lax.fori_loop(0, C // 2, row, 0)
        ocp[s] = pltpu.async_copy(
            bufs_a[s], out_hbm.at[b, pl.ds(s0 + c * C, C)], sems_o[s])
    for s in range(NSLOT):
        if ocp[s] is not None:
            ocp[s].wait()


def kernel(tokens, positions, tok_table, pos_table):
    return _embed_sc(tokens, positions, tok_table, pos_table)
